# baseline (speedup 1.0000x reference)
import functools

import jax
import jax.numpy as jnp
from jax import lax
from jax.experimental import pallas as pl
from jax.experimental.pallas import tpu as pltpu
from jax.experimental.pallas import tpu_sc as plsc

N = 10000
E = 160000
C = 256
HC = C // 2

NC = 2
NS = 16
CHUNK = 128
EPT = 10368
NCHUNK = EPT // CHUNK
E_PAD = NS * EPT
ACC_ROWS = 10048
ZROWS = ACC_ROWS // NS

BLK = 1000
GRID = N // BLK


def _pre_body(pe_ref, xf_ref, wpe_ref, bpe_ref, wx_ref, bx_ref, h_ref):
    eps = 1e-5

    def bn(a):
        m = jnp.mean(a, axis=0)
        v = jnp.mean(a * a, axis=0) - m * m
        return (a - m) / jnp.sqrt(v + eps)

    pe_n = bn(pe_ref[...])
    xn = bn(xf_ref[...])
    t = bn(jnp.dot(pe_n, wpe_ref[...], preferred_element_type=jnp.float32)
           + bpe_ref[...])
    wx = wx_ref[...]
    h_lin = (jnp.dot(xn, wx[:48], preferred_element_type=jnp.float32)
             + jnp.dot(t, wx[48:], preferred_element_type=jnp.float32)
             + bx_ref[...])
    h_ref[...] = bn(h_lin)


def _tc_pre(pe, x_flat, wpe, bpe, wx, bx):
    return pl.pallas_call(
        _pre_body,
        out_shape=jax.ShapeDtypeStruct((N, C), jnp.float32),
    )(pe, x_flat, wpe, bpe, wx, bx)


def _g_body(h_ref, ea_ref, g_ref):
    h = h_ref[...]
    ea = ea_ref[...]
    for c in range(2):
        hc = h[:, c * HC:(c + 1) * HC]
        for a in range(2):
            g_ref[c, a] = jnp.maximum(hc + ea[a, c * HC:(c + 1) * HC], 0.0)


def _tc_gbuild(h, ea):
    return pl.pallas_call(
        _g_body,
        grid=(GRID,),
        in_specs=[
            pl.BlockSpec((BLK, C), lambda i: (i, 0)),
            pl.BlockSpec((2, C), lambda i: (0, 0)),
        ],
        out_specs=pl.BlockSpec((2, 2, BLK, HC), lambda i: (0, 0, i, 0)),
        out_shape=jax.ShapeDtypeStruct((2, 2, N, HC), jnp.float32),
    )(h, ea)


def _layer_body(with_g, h_ref, a_ref, w1_ref, b1_ref, w2_ref, b2_ref, ea_ref,
                hn_ref, g_ref=None):
    z = h_ref[...] + jnp.concatenate([a_ref[0], a_ref[1]], axis=-1)
    t = jnp.maximum(
        jnp.dot(z, w1_ref[...], preferred_element_type=jnp.float32)
        + b1_ref[...], 0.0)
    hn = (jnp.dot(t, w2_ref[...], preferred_element_type=jnp.float32)
          + b2_ref[...])
    hn_ref[...] = hn
    if with_g:
        ea = ea_ref[...]
        for c in range(2):
            hc = hn[:, c * HC:(c + 1) * HC]
            for a in range(2):
                g_ref[c, a] = jnp.maximum(hc + ea[a, c * HC:(c + 1) * HC], 0.0)


def _tc_layer(h, aggr, w1, b1, w2, b2, ea, with_g):
    out_shape = [jax.ShapeDtypeStruct((N, C), jnp.float32)]
    out_specs = [pl.BlockSpec((BLK, C), lambda i: (i, 0))]
    if with_g:
        out_shape.append(jax.ShapeDtypeStruct((2, 2, N, HC), jnp.float32))
        out_specs.append(pl.BlockSpec((2, 2, BLK, HC), lambda i: (0, 0, i, 0)))
    res = pl.pallas_call(
        functools.partial(_layer_body, with_g),
        grid=(GRID,),
        in_specs=[
            pl.BlockSpec((BLK, C), lambda i: (i, 0)),
            pl.BlockSpec((2, BLK, HC), lambda i: (0, i, 0)),
            pl.BlockSpec((C, C), lambda i: (0, 0)),
            pl.BlockSpec((C,), lambda i: (0,)),
            pl.BlockSpec((C, C), lambda i: (0, 0)),
            pl.BlockSpec((C,), lambda i: (0,)),
            pl.BlockSpec((2, C), lambda i: (0, 0)),
        ],
        out_specs=out_specs,
        out_shape=out_shape,
    )(h, aggr, w1, b1, w2, b2, ea)
    return res if with_g else res[0]


NBUF = 3
NITER = NCHUNK // NBUF


def _sc_body(g_hbm, gidx_hbm, didx_hbm, zeros_hbm, out_hbm,
             gidx_v, didx_v, rows_v, acc, gsems, isems):
    c = lax.axis_index("c")
    s = lax.axis_index("s")

    pltpu.sync_copy(zeros_hbm, acc.at[pl.ds(s * ZROWS, ZROWS)])
    for b in range(NBUF):
        pltpu.sync_copy(gidx_hbm.at[c, s, b], gidx_v.at[b])
        pltpu.sync_copy(didx_hbm.at[s, b], didx_v.at[b])
    plsc.subcore_barrier()

    def body(i, carry):
        j = i * NBUF
        cps = []
        for b in range(NBUF):
            @pl.when(i > 0)
            def _():
                pltpu.make_async_copy(gidx_hbm.at[c, s, j + b],
                                      gidx_v.at[b], isems.at[b]).wait()
                pltpu.make_async_copy(didx_hbm.at[s, j + b],
                                      didx_v.at[b], isems.at[b]).wait()
            cps.append(pltpu.async_copy(g_hbm.at[gidx_v.at[b]],
                                        rows_v.at[b], gsems.at[b]))
        for b in range(NBUF):
            cps[b].wait()
            pltpu.sync_copy(rows_v.at[b], acc.at[didx_v.at[b]], add=True)

            @pl.when(i + 1 < NITER)
            def _():
                pltpu.async_copy(gidx_hbm.at[c, s, j + NBUF + b],
                                 gidx_v.at[b], isems.at[b])
                pltpu.async_copy(didx_hbm.at[s, j + NBUF + b],
                                 didx_v.at[b], isems.at[b])
        return carry

    lax.fori_loop(0, NITER, body, 0)
    plsc.subcore_barrier()

    r0 = 624

    @pl.when(s < NS - 1)
    def _():
        pltpu.sync_copy(acc.at[pl.ds(s * r0, r0)],
                        out_hbm.at[c, pl.ds(s * r0, r0)])

    @pl.when(s == NS - 1)
    def _():
        pltpu.sync_copy(acc.at[pl.ds((NS - 1) * r0, N - (NS - 1) * r0)],
                        out_hbm.at[c, pl.ds((NS - 1) * r0, N - (NS - 1) * r0)])


def _sc_mp(g_flat, gidx, didx, zeros):
    mesh = plsc.VectorSubcoreMesh(core_axis_name="c", subcore_axis_name="s",
                                  num_cores=NC, num_subcores=NS)
    f = pl.kernel(
        _sc_body,
        out_type=jax.ShapeDtypeStruct((2, N, HC), jnp.float32),
        mesh=mesh,
        scratch_types=[
            pltpu.VMEM((NBUF, CHUNK), jnp.int32),
            pltpu.VMEM((NBUF, CHUNK), jnp.int32),
            pltpu.VMEM((NBUF, CHUNK, HC), jnp.float32),
            pltpu.VMEM_SHARED((ACC_ROWS, HC), jnp.float32),
            pltpu.SemaphoreType.DMA((NBUF,)),
            pltpu.SemaphoreType.DMA((NBUF,)),
        ],
    )
    return f(g_flat, gidx, didx, zeros)


def kernel(x, pe, params, edge_index, edge_attr, batch):
    del batch
    x_flat = x.reshape(N, -1)

    src = edge_index[0]
    dst = edge_index[1]
    base = edge_attr * N + src
    base = jnp.pad(base, (0, E_PAD - E))
    gidx = jnp.stack([base, base + 2 * N]).reshape(2, NS, NCHUNK, CHUNK)
    didx = jnp.pad(dst, (0, E_PAD - E), constant_values=N)
    didx = didx.reshape(NS, NCHUNK, CHUNK)
    zeros = jnp.zeros((ZROWS, HC), jnp.float32)

    p = params
    ea = p['edge_table']
    h = _tc_pre(pe, x_flat, p['W_pe'], p['b_pe'], p['W_x'], p['b_x'])
    g = _tc_gbuild(h, ea)
    for li, lp in enumerate(p['layers']):
        g_flat = g.reshape(4 * N, HC)
        aggr = _sc_mp(g_flat, gidx, didx, zeros)
        last = li == len(p['layers']) - 1
        if last:
            h = _tc_layer(h, aggr, lp['W1'], lp['b1'], lp['W2'], lp['b2'],
                          ea, with_g=False)
        else:
            h, g = _tc_layer(h, aggr, lp['W1'], lp['b1'], lp['W2'], lp['b2'],
                             ea, with_g=True)
    return h


# speedup vs baseline: 1.3051x; 1.3051x over previous
import functools

import jax
import jax.numpy as jnp
from jax import lax
from jax.experimental import pallas as pl
from jax.experimental.pallas import tpu as pltpu
from jax.experimental.pallas import tpu_sc as plsc

N = 10000
E = 160000
C = 256
HC = C // 2

NC = 2
NS = 16
CHUNK = 64
EPT = 10240
NCHUNK = EPT // CHUNK
E_PAD = NS * EPT
ACC_ROWS = 10240
ZROWS = ACC_ROWS // NS

BLK = 1000
GRID = N // BLK


def _pre_body(pe_ref, xf_ref, wpe_ref, bpe_ref, wx_ref, bx_ref, h_ref):
    eps = 1e-5

    def bn(a):
        m = jnp.mean(a, axis=0)
        v = jnp.mean(a * a, axis=0) - m * m
        return (a - m) / jnp.sqrt(v + eps)

    pe_n = bn(pe_ref[...])
    xn = bn(xf_ref[...])
    t = bn(jnp.dot(pe_n, wpe_ref[...], preferred_element_type=jnp.float32)
           + bpe_ref[...])
    wx = wx_ref[...]
    h_lin = (jnp.dot(xn, wx[:48], preferred_element_type=jnp.float32)
             + jnp.dot(t, wx[48:], preferred_element_type=jnp.float32)
             + bx_ref[...])
    h_ref[...] = bn(h_lin)


def _tc_pre(pe, x_flat, wpe, bpe, wx, bx):
    return pl.pallas_call(
        _pre_body,
        out_shape=jax.ShapeDtypeStruct((N, C), jnp.float32),
    )(pe, x_flat, wpe, bpe, wx, bx)


def _g_body(h_ref, ea_ref, g_ref):
    h = h_ref[...]
    ea = ea_ref[...]
    for c in range(2):
        hc = h[:, c * HC:(c + 1) * HC]
        for a in range(2):
            g_ref[c, a] = jnp.maximum(hc + ea[a, c * HC:(c + 1) * HC], 0.0)


def _tc_gbuild(h, ea):
    return pl.pallas_call(
        _g_body,
        grid=(GRID,),
        in_specs=[
            pl.BlockSpec((BLK, C), lambda i: (i, 0)),
            pl.BlockSpec((2, C), lambda i: (0, 0)),
        ],
        out_specs=pl.BlockSpec((2, 2, BLK, HC), lambda i: (0, 0, i, 0)),
        out_shape=jax.ShapeDtypeStruct((2, 2, N, HC), jnp.float32),
    )(h, ea)


def _layer_body(with_g, h_ref, a_ref, w1_ref, b1_ref, w2_ref, b2_ref, ea_ref,
                hn_ref, g_ref=None):
    z = h_ref[...] + jnp.concatenate([a_ref[0], a_ref[1]], axis=-1)
    t = jnp.maximum(
        jnp.dot(z, w1_ref[...], preferred_element_type=jnp.float32)
        + b1_ref[...], 0.0)
    hn = (jnp.dot(t, w2_ref[...], preferred_element_type=jnp.float32)
          + b2_ref[...])
    hn_ref[...] = hn
    if with_g:
        ea = ea_ref[...]
        for c in range(2):
            hc = hn[:, c * HC:(c + 1) * HC]
            for a in range(2):
                g_ref[c, a] = jnp.maximum(hc + ea[a, c * HC:(c + 1) * HC], 0.0)


def _tc_layer(h, aggr, w1, b1, w2, b2, ea, with_g):
    out_shape = [jax.ShapeDtypeStruct((N, C), jnp.float32)]
    out_specs = [pl.BlockSpec((BLK, C), lambda i: (i, 0))]
    if with_g:
        out_shape.append(jax.ShapeDtypeStruct((2, 2, N, HC), jnp.float32))
        out_specs.append(pl.BlockSpec((2, 2, BLK, HC), lambda i: (0, 0, i, 0)))
    res = pl.pallas_call(
        functools.partial(_layer_body, with_g),
        grid=(GRID,),
        in_specs=[
            pl.BlockSpec((BLK, C), lambda i: (i, 0)),
            pl.BlockSpec((2, BLK, HC), lambda i: (0, i, 0)),
            pl.BlockSpec((C, C), lambda i: (0, 0)),
            pl.BlockSpec((C,), lambda i: (0,)),
            pl.BlockSpec((C, C), lambda i: (0, 0)),
            pl.BlockSpec((C,), lambda i: (0,)),
            pl.BlockSpec((2, C), lambda i: (0, 0)),
        ],
        out_specs=out_specs,
        out_shape=out_shape,
    )(h, aggr, w1, b1, w2, b2, ea)
    return res if with_g else res[0]


NBUF = 4
NITER = NCHUNK // NBUF


def _sc_body(g_hbm, gidx_hbm, didx_hbm, zeros_hbm, out_hbm,
             gidx_v, didx_v, rows_v, acc, gsems, isems):
    c = lax.axis_index("c")
    s = lax.axis_index("s")

    pltpu.sync_copy(zeros_hbm, acc.at[pl.ds(s * ZROWS, ZROWS)])
    for b in range(NBUF):
        pltpu.sync_copy(gidx_hbm.at[c, s, b], gidx_v.at[b])
        pltpu.sync_copy(didx_hbm.at[s, b], didx_v.at[b])
    plsc.subcore_barrier()

    def body(i, carry):
        j = i * NBUF
        cps = []
        for b in range(NBUF):
            @pl.when(i > 0)
            def _():
                pltpu.make_async_copy(gidx_hbm.at[c, s, j + b],
                                      gidx_v.at[b], isems.at[b]).wait()
                pltpu.make_async_copy(didx_hbm.at[s, j + b],
                                      didx_v.at[b], isems.at[b]).wait()
            cps.append(pltpu.async_copy(g_hbm.at[gidx_v.at[b]],
                                        rows_v.at[b], gsems.at[b]))
        for b in range(NBUF):
            cps[b].wait()
            pltpu.sync_copy(rows_v.at[b], acc.at[didx_v.at[b]], add=True)

            @pl.when(i + 1 < NITER)
            def _():
                pltpu.async_copy(gidx_hbm.at[c, s, j + NBUF + b],
                                 gidx_v.at[b], isems.at[b])
                pltpu.async_copy(didx_hbm.at[s, j + NBUF + b],
                                 didx_v.at[b], isems.at[b])
        return carry

    lax.fori_loop(0, NITER, body, 0)
    plsc.subcore_barrier()

    r0 = 624

    @pl.when(s < NS - 1)
    def _():
        pltpu.sync_copy(acc.at[pl.ds(s * r0, r0)],
                        out_hbm.at[c, pl.ds(s * r0, r0)])

    @pl.when(s == NS - 1)
    def _():
        pltpu.sync_copy(acc.at[pl.ds((NS - 1) * r0, N - (NS - 1) * r0)],
                        out_hbm.at[c, pl.ds((NS - 1) * r0, N - (NS - 1) * r0)])


def _sc_mp(g_flat, gidx, didx, zeros):
    mesh = plsc.VectorSubcoreMesh(core_axis_name="c", subcore_axis_name="s",
                                  num_cores=NC, num_subcores=NS)
    f = pl.kernel(
        _sc_body,
        out_type=jax.ShapeDtypeStruct((2, N, HC), jnp.float32),
        mesh=mesh,
        scratch_types=[
            pltpu.VMEM((NBUF, CHUNK), jnp.int32),
            pltpu.VMEM((NBUF, CHUNK), jnp.int32),
            pltpu.VMEM((NBUF, CHUNK, HC), jnp.float32),
            pltpu.VMEM_SHARED((ACC_ROWS, HC), jnp.float32),
            pltpu.SemaphoreType.DMA((NBUF,)),
            pltpu.SemaphoreType.DMA((NBUF,)),
        ],
    )
    return f(g_flat, gidx, didx, zeros)


def kernel(x, pe, params, edge_index, edge_attr, batch):
    del batch
    x_flat = x.reshape(N, -1)

    src = edge_index[0]
    dst = edge_index[1]
    base = edge_attr * N + src
    base = jnp.pad(base, (0, E_PAD - E))
    gidx = jnp.stack([base, base + 2 * N]).reshape(2, NS, NCHUNK, CHUNK)
    didx = jnp.pad(dst, (0, E_PAD - E), constant_values=N)
    didx = didx.reshape(NS, NCHUNK, CHUNK)
    zeros = jnp.zeros((ZROWS, HC), jnp.float32)

    p = params
    ea = p['edge_table']
    h = _tc_pre(pe, x_flat, p['W_pe'], p['b_pe'], p['W_x'], p['b_x'])
    g = _tc_gbuild(h, ea)
    for li, lp in enumerate(p['layers']):
        g_flat = g.reshape(4 * N, HC)
        aggr = _sc_mp(g_flat, gidx, didx, zeros)
        last = li == len(p['layers']) - 1
        if last:
            h = _tc_layer(h, aggr, lp['W1'], lp['b1'], lp['W2'], lp['b2'],
                          ea, with_g=False)
        else:
            h, g = _tc_layer(h, aggr, lp['W1'], lp['b1'], lp['W2'], lp['b2'],
                             ea, with_g=True)
    return h


# speedup vs baseline: 1.3798x; 1.0573x over previous
import functools

import jax
import jax.numpy as jnp
from jax import lax
from jax.experimental import pallas as pl
from jax.experimental.pallas import tpu as pltpu
from jax.experimental.pallas import tpu_sc as plsc

N = 10000
E = 160000
C = 256
HC = C // 2

NC = 2
NS = 16
CHUNK = 32
EPT = 10240
NCHUNK = EPT // CHUNK
E_PAD = NS * EPT
ACC_ROWS = 10240
ZROWS = ACC_ROWS // NS

BLK = 1000
GRID = N // BLK


def _pre_body(pe_ref, xf_ref, wpe_ref, bpe_ref, wx_ref, bx_ref, h_ref):
    eps = 1e-5

    def bn(a):
        m = jnp.mean(a, axis=0)
        v = jnp.mean(a * a, axis=0) - m * m
        return (a - m) / jnp.sqrt(v + eps)

    pe_n = bn(pe_ref[...])
    xn = bn(xf_ref[...])
    t = bn(jnp.dot(pe_n, wpe_ref[...], preferred_element_type=jnp.float32)
           + bpe_ref[...])
    wx = wx_ref[...]
    h_lin = (jnp.dot(xn, wx[:48], preferred_element_type=jnp.float32)
             + jnp.dot(t, wx[48:], preferred_element_type=jnp.float32)
             + bx_ref[...])
    h_ref[...] = bn(h_lin)


def _tc_pre(pe, x_flat, wpe, bpe, wx, bx):
    return pl.pallas_call(
        _pre_body,
        out_shape=jax.ShapeDtypeStruct((N, C), jnp.float32),
    )(pe, x_flat, wpe, bpe, wx, bx)


def _g_body(h_ref, ea_ref, g_ref):
    h = h_ref[...]
    ea = ea_ref[...]
    for c in range(2):
        hc = h[:, c * HC:(c + 1) * HC]
        for a in range(2):
            g_ref[c, a] = jnp.maximum(hc + ea[a, c * HC:(c + 1) * HC], 0.0)


def _tc_gbuild(h, ea):
    return pl.pallas_call(
        _g_body,
        grid=(GRID,),
        in_specs=[
            pl.BlockSpec((BLK, C), lambda i: (i, 0)),
            pl.BlockSpec((2, C), lambda i: (0, 0)),
        ],
        out_specs=pl.BlockSpec((2, 2, BLK, HC), lambda i: (0, 0, i, 0)),
        out_shape=jax.ShapeDtypeStruct((2, 2, N, HC), jnp.float32),
    )(h, ea)


def _layer_body(with_g, h_ref, a_ref, w1_ref, b1_ref, w2_ref, b2_ref, ea_ref,
                hn_ref, g_ref=None):
    z = h_ref[...] + jnp.concatenate([a_ref[0], a_ref[1]], axis=-1)
    t = jnp.maximum(
        jnp.dot(z, w1_ref[...], preferred_element_type=jnp.float32)
        + b1_ref[...], 0.0)
    hn = (jnp.dot(t, w2_ref[...], preferred_element_type=jnp.float32)
          + b2_ref[...])
    hn_ref[...] = hn
    if with_g:
        ea = ea_ref[...]
        for c in range(2):
            hc = hn[:, c * HC:(c + 1) * HC]
            for a in range(2):
                g_ref[c, a] = jnp.maximum(hc + ea[a, c * HC:(c + 1) * HC], 0.0)


def _tc_layer(h, aggr, w1, b1, w2, b2, ea, with_g):
    out_shape = [jax.ShapeDtypeStruct((N, C), jnp.float32)]
    out_specs = [pl.BlockSpec((BLK, C), lambda i: (i, 0))]
    if with_g:
        out_shape.append(jax.ShapeDtypeStruct((2, 2, N, HC), jnp.float32))
        out_specs.append(pl.BlockSpec((2, 2, BLK, HC), lambda i: (0, 0, i, 0)))
    res = pl.pallas_call(
        functools.partial(_layer_body, with_g),
        grid=(GRID,),
        in_specs=[
            pl.BlockSpec((BLK, C), lambda i: (i, 0)),
            pl.BlockSpec((2, BLK, HC), lambda i: (0, i, 0)),
            pl.BlockSpec((C, C), lambda i: (0, 0)),
            pl.BlockSpec((C,), lambda i: (0,)),
            pl.BlockSpec((C, C), lambda i: (0, 0)),
            pl.BlockSpec((C,), lambda i: (0,)),
            pl.BlockSpec((2, C), lambda i: (0, 0)),
        ],
        out_specs=out_specs,
        out_shape=out_shape,
    )(h, aggr, w1, b1, w2, b2, ea)
    return res if with_g else res[0]


NBUF = 8
NITER = NCHUNK // NBUF


def _sc_body(g_hbm, gidx_hbm, didx_hbm, zeros_hbm, out_hbm,
             gidx_v, didx_v, rows_v, acc, gsems, isems):
    c = lax.axis_index("c")
    s = lax.axis_index("s")

    pltpu.sync_copy(zeros_hbm, acc.at[pl.ds(s * ZROWS, ZROWS)])
    for b in range(NBUF):
        pltpu.sync_copy(gidx_hbm.at[c, s, b], gidx_v.at[b])
        pltpu.sync_copy(didx_hbm.at[s, b], didx_v.at[b])
    plsc.subcore_barrier()

    def body(i, carry):
        j = i * NBUF
        cps = []
        for b in range(NBUF):
            @pl.when(i > 0)
            def _():
                pltpu.make_async_copy(gidx_hbm.at[c, s, j + b],
                                      gidx_v.at[b], isems.at[b]).wait()
                pltpu.make_async_copy(didx_hbm.at[s, j + b],
                                      didx_v.at[b], isems.at[b]).wait()
            cps.append(pltpu.async_copy(g_hbm.at[gidx_v.at[b]],
                                        rows_v.at[b], gsems.at[b]))
        for b in range(NBUF):
            cps[b].wait()
            pltpu.sync_copy(rows_v.at[b], acc.at[didx_v.at[b]], add=True)

            @pl.when(i + 1 < NITER)
            def _():
                pltpu.async_copy(gidx_hbm.at[c, s, j + NBUF + b],
                                 gidx_v.at[b], isems.at[b])
                pltpu.async_copy(didx_hbm.at[s, j + NBUF + b],
                                 didx_v.at[b], isems.at[b])
        return carry

    lax.fori_loop(0, NITER, body, 0)
    plsc.subcore_barrier()

    r0 = 624

    @pl.when(s < NS - 1)
    def _():
        pltpu.sync_copy(acc.at[pl.ds(s * r0, r0)],
                        out_hbm.at[c, pl.ds(s * r0, r0)])

    @pl.when(s == NS - 1)
    def _():
        pltpu.sync_copy(acc.at[pl.ds((NS - 1) * r0, N - (NS - 1) * r0)],
                        out_hbm.at[c, pl.ds((NS - 1) * r0, N - (NS - 1) * r0)])


def _sc_mp(g_flat, gidx, didx, zeros):
    mesh = plsc.VectorSubcoreMesh(core_axis_name="c", subcore_axis_name="s",
                                  num_cores=NC, num_subcores=NS)
    f = pl.kernel(
        _sc_body,
        out_type=jax.ShapeDtypeStruct((2, N, HC), jnp.float32),
        mesh=mesh,
        scratch_types=[
            pltpu.VMEM((NBUF, CHUNK), jnp.int32),
            pltpu.VMEM((NBUF, CHUNK), jnp.int32),
            pltpu.VMEM((NBUF, CHUNK, HC), jnp.float32),
            pltpu.VMEM_SHARED((ACC_ROWS, HC), jnp.float32),
            pltpu.SemaphoreType.DMA((NBUF,)),
            pltpu.SemaphoreType.DMA((NBUF,)),
        ],
    )
    return f(g_flat, gidx, didx, zeros)


def kernel(x, pe, params, edge_index, edge_attr, batch):
    del batch
    x_flat = x.reshape(N, -1)

    src = edge_index[0]
    dst = edge_index[1]
    base = edge_attr * N + src
    base = jnp.pad(base, (0, E_PAD - E))
    gidx = jnp.stack([base, base + 2 * N]).reshape(2, NS, NCHUNK, CHUNK)
    didx = jnp.pad(dst, (0, E_PAD - E), constant_values=N)
    didx = didx.reshape(NS, NCHUNK, CHUNK)
    zeros = jnp.zeros((ZROWS, HC), jnp.float32)

    p = params
    ea = p['edge_table']
    h = _tc_pre(pe, x_flat, p['W_pe'], p['b_pe'], p['W_x'], p['b_x'])
    g = _tc_gbuild(h, ea)
    for li, lp in enumerate(p['layers']):
        g_flat = g.reshape(4 * N, HC)
        aggr = _sc_mp(g_flat, gidx, didx, zeros)
        last = li == len(p['layers']) - 1
        if last:
            h = _tc_layer(h, aggr, lp['W1'], lp['b1'], lp['W2'], lp['b2'],
                          ea, with_g=False)
        else:
            h, g = _tc_layer(h, aggr, lp['W1'], lp['b1'], lp['W2'], lp['b2'],
                             ea, with_g=True)
    return h
